# Initial kernel scaffold; baseline (speedup 1.0000x reference)
import functools

import jax
import jax.numpy as jnp
from jax import lax
from jax.experimental import pallas as pl
from jax.experimental.pallas import tpu as pltpu
from jax.experimental.pallas import tpu_sc as plsc

N_NODES = 10000
IN_DIM = 128
H = 64

NC = 2
NS = 16
NW = NC * NS
CHUNK = 128
N_PAD = 10016
ROWS_PER_TILE = N_PAD // NS
CNT_W = 16

_mesh = plsc.VectorSubcoreMesh(core_axis_name="c", subcore_axis_name="s",
                               num_cores=NC, num_subcores=NS)


def _make_sc_scatter(n_chunks, with_count):
  out_type = [jax.ShapeDtypeStruct((NC, N_PAD, H), jnp.float32)]
  if with_count:
    out_type.append(jax.ShapeDtypeStruct((NC, N_PAD, CNT_W), jnp.float32))

  scratch = [
      pltpu.VMEM((n_chunks, CHUNK), jnp.int32),
      pltpu.VMEM((n_chunks, CHUNK), jnp.int32),
      pltpu.VMEM((CHUNK, H), jnp.float32),
      pltpu.VMEM_SHARED((N_PAD, H), jnp.float32),
      pltpu.SemaphoreType.DMA,
  ]
  if with_count:
    scratch += [
        pltpu.VMEM((CHUNK, CNT_W), jnp.float32),
        pltpu.VMEM_SHARED((N_PAD, CNT_W), jnp.float32),
    ]

  def body(p_hbm, src_hbm, dst_hbm, zeros64, *rest):
    if with_count:
      (ones_hbm, zeros16, acc_out, cnt_out,
       sidx, didx, gbuf, acc_sh, sem, ones_v, cnt_sh) = rest
    else:
      acc_out, sidx, didx, gbuf, acc_sh, sem = rest
    cid = lax.axis_index("c")
    sid = lax.axis_index("s")
    w = cid * NS + sid
    row0 = sid * ROWS_PER_TILE

    pltpu.sync_copy(zeros64.at[pl.ds(row0, ROWS_PER_TILE)],
                    acc_sh.at[pl.ds(row0, ROWS_PER_TILE)])
    if with_count:
      pltpu.sync_copy(zeros16.at[pl.ds(row0, ROWS_PER_TILE)],
                      cnt_sh.at[pl.ds(row0, ROWS_PER_TILE)])
      pltpu.sync_copy(ones_hbm, ones_v)
    pltpu.sync_copy(src_hbm.at[w], sidx)
    pltpu.sync_copy(dst_hbm.at[w], didx)
    plsc.subcore_barrier()

    def chunk_body(j, carry):
      pltpu.async_copy(p_hbm.at[sidx.at[j]], gbuf, sem).wait()
      pltpu.sync_copy(gbuf, acc_sh.at[didx.at[j]], add=True)
      if with_count:
        pltpu.sync_copy(ones_v, cnt_sh.at[didx.at[j]], add=True)
      return carry

    lax.fori_loop(0, n_chunks, chunk_body, 0)
    plsc.subcore_barrier()

    pltpu.sync_copy(acc_sh.at[pl.ds(row0, ROWS_PER_TILE)],
                    acc_out.at[cid, pl.ds(row0, ROWS_PER_TILE)])
    if with_count:
      pltpu.sync_copy(cnt_sh.at[pl.ds(row0, ROWS_PER_TILE)],
                      cnt_out.at[cid, pl.ds(row0, ROWS_PER_TILE)])

  return pl.kernel(body, out_type=out_type, mesh=_mesh,
                   scratch_types=scratch)


def _tc_pre(x_ref, wl_ref, wr_ref, p_ref, r_ref):
  x = x_ref[...]
  p_ref[...] = jnp.dot(x, wl_ref[...], preferred_element_type=jnp.float32)
  r_ref[...] = jnp.dot(x, wr_ref[...], preferred_element_type=jnp.float32)


def _tc_mid(acc_ref, cnt_ref, r1_ref, b1_ref, wl_ref, wr_ref,
            p2_ref, r2_ref, inv_ref):
  cnt = cnt_ref[0] + cnt_ref[1]
  inv = 1.0 / jnp.maximum(cnt, 1.0)
  inv_ref[...] = inv
  agg = acc_ref[0] + acc_ref[1]
  z = jnp.maximum(agg * inv[:, :1] + b1_ref[...] + r1_ref[...], 0.0)
  p2_ref[...] = jnp.dot(z, wl_ref[...], preferred_element_type=jnp.float32)
  r2_ref[...] = jnp.dot(z, wr_ref[...], preferred_element_type=jnp.float32)


def _tc_final(acc_ref, inv_ref, r2_ref, b2_ref, out_ref):
  agg = acc_ref[0] + acc_ref[1]
  out_ref[...] = agg * inv_ref[:, :1] + b2_ref[...] + r2_ref[...]


@jax.jit
def kernel(x, edge_index, W1l, b1l, W1r, W2l, b2l, W2r):
  n_edges = edge_index.shape[1]
  n_chunks = -(-n_edges // (NW * CHUNK))
  e_pad = NW * n_chunks * CHUNK

  src = edge_index[0].astype(jnp.int32)
  dst = edge_index[1].astype(jnp.int32)
  pad = jnp.full((e_pad - n_edges,), N_NODES, dtype=jnp.int32)
  src = jnp.concatenate([src, pad]).reshape(NW, n_chunks, CHUNK)
  dst = jnp.concatenate([dst, pad]).reshape(NW, n_chunks, CHUNK)

  xp = jnp.zeros((N_PAD, IN_DIM), jnp.float32).at[:N_NODES].set(x)
  zeros64 = jnp.zeros((N_PAD, H), jnp.float32)
  zeros16 = jnp.zeros((N_PAD, CNT_W), jnp.float32)
  ones = jnp.ones((CHUNK, CNT_W), jnp.float32)

  p1, r1 = pl.pallas_call(
      _tc_pre,
      out_shape=[jax.ShapeDtypeStruct((N_PAD, H), jnp.float32)] * 2,
  )(xp, W1l.T, W1r.T)

  sc1 = _make_sc_scatter(n_chunks, with_count=True)
  acc1, cnt = sc1(p1, src, dst, zeros64, ones, zeros16)

  p2, r2, inv = pl.pallas_call(
      _tc_mid,
      out_shape=[jax.ShapeDtypeStruct((N_PAD, H), jnp.float32)] * 2
      + [jax.ShapeDtypeStruct((N_PAD, CNT_W), jnp.float32)],
  )(acc1, cnt, r1, b1l.reshape(1, H), W2l.T, W2r.T)

  sc2 = _make_sc_scatter(n_chunks, with_count=False)
  (acc2,) = sc2(p2, src, dst, zeros64)

  out = pl.pallas_call(
      _tc_final,
      out_shape=jax.ShapeDtypeStruct((N_PAD, H), jnp.float32),
  )(acc2, inv, r2, b2l.reshape(1, H))

  return out[:N_NODES]


# speedup vs baseline: 6.6295x; 6.6295x over previous
import functools

import jax
import jax.numpy as jnp
from jax import lax
from jax.experimental import pallas as pl
from jax.experimental.pallas import tpu as pltpu
from jax.experimental.pallas import tpu_sc as plsc

N_NODES = 10000
IN_DIM = 128
H = 64

NC = 2
NS = 16
NW = NC * NS
CHUNK = 128
N_PAD = 10112
ROWS_PER_TILE = N_PAD // NS
CNT_W = 16

def _make_sc_scatter(n_chunks, with_count):
  out_type = [jax.ShapeDtypeStruct((NC, N_PAD, H), jnp.float32)]
  if with_count:
    out_type.append(jax.ShapeDtypeStruct((NC, N_PAD, CNT_W), jnp.float32))

  scratch = [
      pltpu.VMEM((n_chunks, CHUNK), jnp.int32),
      pltpu.VMEM((n_chunks, CHUNK), jnp.int32),
      pltpu.VMEM((CHUNK, H), jnp.float32),
      pltpu.VMEM_SHARED((N_PAD, H), jnp.float32),
      pltpu.SemaphoreType.DMA,
  ]
  if with_count:
    scratch += [
        pltpu.VMEM((CHUNK, CNT_W), jnp.float32),
        pltpu.VMEM_SHARED((N_PAD, CNT_W), jnp.float32),
    ]

  def body(p_hbm, src_hbm, dst_hbm, zeros64, *rest):
    if with_count:
      (ones_hbm, zeros16, acc_out, cnt_out,
       sidx, didx, gbuf, acc_sh, sem, ones_v, cnt_sh) = rest
    else:
      acc_out, sidx, didx, gbuf, acc_sh, sem = rest
    cid = lax.axis_index("c")
    sid = lax.axis_index("s")
    w = cid * NS + sid
    row0 = sid * ROWS_PER_TILE

    pltpu.sync_copy(zeros64.at[pl.ds(row0, ROWS_PER_TILE)],
                    acc_sh.at[pl.ds(row0, ROWS_PER_TILE)])
    if with_count:
      pltpu.sync_copy(zeros16.at[pl.ds(row0, ROWS_PER_TILE)],
                      cnt_sh.at[pl.ds(row0, ROWS_PER_TILE)])
      pltpu.sync_copy(ones_hbm, ones_v)
    pltpu.sync_copy(src_hbm.at[w], sidx)
    pltpu.sync_copy(dst_hbm.at[w], didx)
    plsc.subcore_barrier()

    def chunk_body(j, carry):
      pltpu.async_copy(p_hbm.at[sidx.at[j]], gbuf, sem).wait()
      pltpu.sync_copy(gbuf, acc_sh.at[didx.at[j]], add=True)
      if with_count:
        pltpu.sync_copy(ones_v, cnt_sh.at[didx.at[j]], add=True)
      return carry

    lax.fori_loop(0, n_chunks, chunk_body, 0)
    plsc.subcore_barrier()

    pltpu.sync_copy(acc_sh.at[pl.ds(row0, ROWS_PER_TILE)],
                    acc_out.at[cid, pl.ds(row0, ROWS_PER_TILE)])
    if with_count:
      pltpu.sync_copy(cnt_sh.at[pl.ds(row0, ROWS_PER_TILE)],
                      cnt_out.at[cid, pl.ds(row0, ROWS_PER_TILE)])

  mesh = plsc.VectorSubcoreMesh(core_axis_name="c", subcore_axis_name="s",
                                num_cores=NC, num_subcores=NS)
  return pl.kernel(body, out_type=out_type, mesh=mesh,
                   scratch_types=scratch,
                   compiler_params=pltpu.CompilerParams(
                       use_tc_tiling_on_sc=False))


def _tc_pre(x_ref, wl_ref, wr_ref, p_ref, r_ref):
  x = x_ref[...]
  p_ref[...] = jnp.dot(x, wl_ref[...], preferred_element_type=jnp.float32)
  r_ref[...] = jnp.dot(x, wr_ref[...], preferred_element_type=jnp.float32)


def _tc_mid(acc_ref, cnt_ref, r1_ref, b1_ref, wl_ref, wr_ref,
            p2_ref, r2_ref, inv_ref):
  cnt = cnt_ref[0] + cnt_ref[1]
  inv = 1.0 / jnp.maximum(cnt, 1.0)
  inv_ref[...] = inv
  agg = acc_ref[0] + acc_ref[1]
  z = jnp.maximum(agg * inv[:, :1] + b1_ref[...] + r1_ref[...], 0.0)
  p2_ref[...] = jnp.dot(z, wl_ref[...], preferred_element_type=jnp.float32)
  r2_ref[...] = jnp.dot(z, wr_ref[...], preferred_element_type=jnp.float32)


def _tc_final(acc_ref, inv_ref, r2_ref, b2_ref, out_ref):
  agg = acc_ref[0] + acc_ref[1]
  out_ref[...] = agg * inv_ref[:, :1] + b2_ref[...] + r2_ref[...]


@jax.jit
def kernel(x, edge_index, W1l, b1l, W1r, W2l, b2l, W2r):
  n_edges = edge_index.shape[1]
  n_chunks = -(-n_edges // (NW * CHUNK))
  e_pad = NW * n_chunks * CHUNK

  src = edge_index[0].astype(jnp.int32)
  dst = edge_index[1].astype(jnp.int32)
  pad = jnp.full((e_pad - n_edges,), N_NODES, dtype=jnp.int32)
  src = jnp.concatenate([src, pad]).reshape(NW, n_chunks, CHUNK)
  dst = jnp.concatenate([dst, pad]).reshape(NW, n_chunks, CHUNK)

  xp = jnp.zeros((N_PAD, IN_DIM), jnp.float32).at[:N_NODES].set(x)
  zeros64 = jnp.zeros((N_PAD, H), jnp.float32)
  zeros16 = jnp.zeros((N_PAD, CNT_W), jnp.float32)
  ones = jnp.ones((CHUNK, CNT_W), jnp.float32)

  p1, r1 = pl.pallas_call(
      _tc_pre,
      out_shape=[jax.ShapeDtypeStruct((N_PAD, H), jnp.float32)] * 2,
  )(xp, W1l.T, W1r.T)

  sc1 = _make_sc_scatter(n_chunks, with_count=True)
  acc1, cnt = sc1(p1, src, dst, zeros64, ones, zeros16)

  p2, r2, inv = pl.pallas_call(
      _tc_mid,
      out_shape=[jax.ShapeDtypeStruct((N_PAD, H), jnp.float32)] * 2
      + [jax.ShapeDtypeStruct((N_PAD, CNT_W), jnp.float32)],
  )(acc1, cnt, r1, b1l.reshape(1, H), W2l.T, W2r.T)

  sc2 = _make_sc_scatter(n_chunks, with_count=False)
  (acc2,) = sc2(p2, src, dst, zeros64)

  out = pl.pallas_call(
      _tc_final,
      out_shape=jax.ShapeDtypeStruct((N_PAD, H), jnp.float32),
  )(acc2, inv, r2, b2l.reshape(1, H))

  return out[:N_NODES]


# speedup vs baseline: 14.5636x; 2.1968x over previous
import functools

import jax
import jax.numpy as jnp
from jax import lax
from jax.experimental import pallas as pl
from jax.experimental.pallas import tpu as pltpu
from jax.experimental.pallas import tpu_sc as plsc

N_NODES = 10000
IN_DIM = 128
H = 64

NC = 2
NS = 16
NW = NC * NS
CHUNK = 128
N_PAD = 10112
ROWS_PER_TILE = N_PAD // NS
CNT_W = 16

NBUF = 6
LEAD = 3


def _make_sc_scatter(n_chunks, with_count):
  assert n_chunks % NBUF == 0 and n_chunks >= NBUF
  out_type = [jax.ShapeDtypeStruct((NC, N_PAD, H), jnp.float32)]
  if with_count:
    out_type.append(jax.ShapeDtypeStruct((NC, N_PAD, CNT_W), jnp.float32))

  scratch = [
      pltpu.VMEM((n_chunks, CHUNK), jnp.int32),
      pltpu.VMEM((n_chunks, CHUNK), jnp.int32),
      pltpu.VMEM_SHARED((N_PAD, H), jnp.float32),
  ]
  scratch += [pltpu.VMEM((CHUNK, H), jnp.float32)] * NBUF
  scratch += [pltpu.SemaphoreType.DMA] * NBUF
  scratch += [pltpu.SemaphoreType.DMA] * NBUF
  if with_count:
    scratch += [
        pltpu.VMEM((CHUNK, CNT_W), jnp.float32),
        pltpu.VMEM_SHARED((N_PAD, CNT_W), jnp.float32),
    ]
    scratch += [pltpu.SemaphoreType.DMA] * NBUF

  def body(p_hbm, src_hbm, dst_hbm, zeros64, *rest):
    if with_count:
      ones_hbm, zeros16, acc_out, cnt_out = rest[:4]
      rest = rest[4:]
    else:
      acc_out = rest[0]
      rest = rest[1:]
    sidx, didx, acc_sh = rest[:3]
    gbufs = rest[3:3 + NBUF]
    semg = rest[3 + NBUF:3 + 2 * NBUF]
    sems = rest[3 + 2 * NBUF:3 + 3 * NBUF]
    if with_count:
      ones_v, cnt_sh = rest[3 + 3 * NBUF:3 + 3 * NBUF + 2]
      semc = rest[3 + 3 * NBUF + 2:]
    cid = lax.axis_index("c")
    sid = lax.axis_index("s")
    w = cid * NS + sid
    row0 = sid * ROWS_PER_TILE

    pltpu.sync_copy(zeros64.at[pl.ds(row0, ROWS_PER_TILE)],
                    acc_sh.at[pl.ds(row0, ROWS_PER_TILE)])
    if with_count:
      pltpu.sync_copy(zeros16.at[pl.ds(row0, ROWS_PER_TILE)],
                      cnt_sh.at[pl.ds(row0, ROWS_PER_TILE)])
      pltpu.sync_copy(ones_hbm, ones_v)
    pltpu.sync_copy(src_hbm.at[w], sidx)
    pltpu.sync_copy(dst_hbm.at[w], didx)
    plsc.subcore_barrier()

    def wait_scatter(slot):
      pltpu.make_async_copy(gbufs[slot], acc_sh.at[didx.at[0]],
                            sems[slot]).wait()
      if with_count:
        pltpu.make_async_copy(ones_v, cnt_sh.at[didx.at[0]],
                              semc[slot]).wait()

    for b in range(LEAD):
      pltpu.async_copy(p_hbm.at[sidx.at[b]], gbufs[b], semg[b])

    def group_body(g, carry):
      j0 = g * NBUF
      for b in range(NBUF):
        jj = j0 + b
        bw = (b + LEAD) % NBUF
        if b < LEAD:
          @pl.when(g > 0)
          def _():
            wait_scatter(bw)
        else:
          wait_scatter(bw)
        @pl.when(jj + LEAD < n_chunks)
        def _():
          pltpu.async_copy(p_hbm.at[sidx.at[jj + LEAD]], gbufs[bw], semg[bw])
        pltpu.make_async_copy(p_hbm.at[sidx.at[0]], gbufs[b], semg[b]).wait()
        pltpu.async_copy(gbufs[b], acc_sh.at[didx.at[jj]], sems[b], add=True)
        if with_count:
          pltpu.async_copy(ones_v, cnt_sh.at[didx.at[jj]], semc[b], add=True)
      return carry

    lax.fori_loop(0, n_chunks // NBUF, group_body, 0)
    for b in range(LEAD, NBUF):
      wait_scatter(b)
    plsc.subcore_barrier()

    pltpu.sync_copy(acc_sh.at[pl.ds(row0, ROWS_PER_TILE)],
                    acc_out.at[cid, pl.ds(row0, ROWS_PER_TILE)])
    if with_count:
      pltpu.sync_copy(cnt_sh.at[pl.ds(row0, ROWS_PER_TILE)],
                      cnt_out.at[cid, pl.ds(row0, ROWS_PER_TILE)])

  mesh = plsc.VectorSubcoreMesh(core_axis_name="c", subcore_axis_name="s",
                                num_cores=NC, num_subcores=NS)
  return pl.kernel(body, out_type=out_type, mesh=mesh,
                   scratch_types=scratch,
                   compiler_params=pltpu.CompilerParams(
                       use_tc_tiling_on_sc=False))


def _tc_pre(x_ref, wl_ref, wr_ref, p_ref, r_ref):
  x = x_ref[...]
  p_ref[...] = jnp.dot(x, wl_ref[...], preferred_element_type=jnp.float32)
  r_ref[...] = jnp.dot(x, wr_ref[...], preferred_element_type=jnp.float32)


def _tc_mid(acc_ref, cnt_ref, r1_ref, b1_ref, wl_ref, wr_ref,
            p2_ref, r2_ref, inv_ref):
  cnt = cnt_ref[0] + cnt_ref[1]
  inv = 1.0 / jnp.maximum(cnt, 1.0)
  inv_ref[...] = inv
  agg = acc_ref[0] + acc_ref[1]
  z = jnp.maximum(agg * inv[:, :1] + b1_ref[...] + r1_ref[...], 0.0)
  p2_ref[...] = jnp.dot(z, wl_ref[...], preferred_element_type=jnp.float32)
  r2_ref[...] = jnp.dot(z, wr_ref[...], preferred_element_type=jnp.float32)


def _tc_final(acc_ref, inv_ref, r2_ref, b2_ref, out_ref):
  agg = acc_ref[0] + acc_ref[1]
  out_ref[...] = agg * inv_ref[:, :1] + b2_ref[...] + r2_ref[...]


@jax.jit
def kernel(x, edge_index, W1l, b1l, W1r, W2l, b2l, W2r):
  n_edges = edge_index.shape[1]
  n_chunks = -(-n_edges // (NW * CHUNK))
  n_chunks = -(-n_chunks // NBUF) * NBUF
  e_pad = NW * n_chunks * CHUNK

  src = edge_index[0].astype(jnp.int32)
  dst = edge_index[1].astype(jnp.int32)
  pad = N_NODES + jnp.arange(e_pad - n_edges, dtype=jnp.int32) % (
      N_PAD - N_NODES)
  src = jnp.concatenate([src, pad]).reshape(NW, n_chunks, CHUNK)
  dst = jnp.concatenate([dst, pad]).reshape(NW, n_chunks, CHUNK)

  xp = jnp.zeros((N_PAD, IN_DIM), jnp.float32).at[:N_NODES].set(x)
  zeros64 = jnp.zeros((N_PAD, H), jnp.float32)
  zeros16 = jnp.zeros((N_PAD, CNT_W), jnp.float32)
  ones = jnp.ones((CHUNK, CNT_W), jnp.float32)

  p1, r1 = pl.pallas_call(
      _tc_pre,
      out_shape=[jax.ShapeDtypeStruct((N_PAD, H), jnp.float32)] * 2,
  )(xp, W1l.T, W1r.T)

  sc1 = _make_sc_scatter(n_chunks, with_count=True)
  acc1, cnt = sc1(p1, src, dst, zeros64, ones, zeros16)

  p2, r2, inv = pl.pallas_call(
      _tc_mid,
      out_shape=[jax.ShapeDtypeStruct((N_PAD, H), jnp.float32)] * 2
      + [jax.ShapeDtypeStruct((N_PAD, CNT_W), jnp.float32)],
  )(acc1, cnt, r1, b1l.reshape(1, H), W2l.T, W2r.T)

  sc2 = _make_sc_scatter(n_chunks, with_count=False)
  (acc2,) = sc2(p2, src, dst, zeros64)

  out = pl.pallas_call(
      _tc_final,
      out_shape=jax.ShapeDtypeStruct((N_PAD, H), jnp.float32),
  )(acc2, inv, r2, b2l.reshape(1, H))

  return out[:N_NODES]


# speedup vs baseline: 16.4634x; 1.1305x over previous
import jax
import jax.numpy as jnp
import numpy as np
from jax import lax
from jax.experimental import pallas as pl
from jax.experimental.pallas import tpu as pltpu
from jax.experimental.pallas import tpu_sc as plsc

N_NODES = 10000
IN_DIM = 128
H = 64

NC = 2
NS = 16
NW = NC * NS
CHUNK = 128
N_PAD = 10112
ROWS_PER_TILE = N_PAD // NS
CNT_W = 16

NBUF = 6
LEAD = 3


def _make_sc_scatter(n_chunks, with_count):
  assert n_chunks % NBUF == 0 and n_chunks >= NBUF
  out_type = [jax.ShapeDtypeStruct((N_PAD, NC * H), jnp.float32)]
  if with_count:
    out_type.append(jax.ShapeDtypeStruct((N_PAD, 128), jnp.float32))

  scratch = [
      pltpu.VMEM((n_chunks, CHUNK), jnp.int32),
      pltpu.VMEM((n_chunks, CHUNK), jnp.int32),
      pltpu.VMEM_SHARED((N_PAD, H), jnp.float32),
  ]
  scratch += [pltpu.VMEM((CHUNK, H), jnp.float32)] * NBUF
  scratch += [pltpu.SemaphoreType.DMA] * NBUF
  scratch += [pltpu.SemaphoreType.DMA] * NBUF
  if with_count:
    scratch += [
        pltpu.VMEM((CHUNK, CNT_W), jnp.float32),
        pltpu.VMEM_SHARED((N_PAD, CNT_W), jnp.float32),
    ]
    scratch += [pltpu.SemaphoreType.DMA] * NBUF

  def body(p_hbm, src_hbm, dst_hbm, zeros64, *rest):
    if with_count:
      ones_hbm, zeros16, acc_out, cnt_out = rest[:4]
      rest = rest[4:]
    else:
      acc_out = rest[0]
      rest = rest[1:]
    sidx, didx, acc_sh = rest[:3]
    gbufs = rest[3:3 + NBUF]
    semg = rest[3 + NBUF:3 + 2 * NBUF]
    sems = rest[3 + 2 * NBUF:3 + 3 * NBUF]
    if with_count:
      ones_v, cnt_sh = rest[3 + 3 * NBUF:3 + 3 * NBUF + 2]
      semc = rest[3 + 3 * NBUF + 2:]
    cid = lax.axis_index("c")
    sid = lax.axis_index("s")
    w = cid * NS + sid
    row0 = sid * ROWS_PER_TILE

    pltpu.sync_copy(zeros64.at[pl.ds(row0, ROWS_PER_TILE)],
                    acc_sh.at[pl.ds(row0, ROWS_PER_TILE)])
    if with_count:
      pltpu.sync_copy(zeros16.at[pl.ds(row0, ROWS_PER_TILE)],
                      cnt_sh.at[pl.ds(row0, ROWS_PER_TILE)])
      pltpu.sync_copy(ones_hbm, ones_v)
    pltpu.sync_copy(src_hbm.at[w], sidx)
    pltpu.sync_copy(dst_hbm.at[w], didx)
    plsc.subcore_barrier()

    def wait_scatter(slot):
      pltpu.make_async_copy(gbufs[slot], acc_sh.at[didx.at[0]],
                            sems[slot]).wait()
      if with_count:
        pltpu.make_async_copy(ones_v, cnt_sh.at[didx.at[0]],
                              semc[slot]).wait()

    for b in range(LEAD):
      pltpu.async_copy(p_hbm.at[sidx.at[b]], gbufs[b], semg[b])

    def group_body(g, carry):
      j0 = g * NBUF
      for b in range(NBUF):
        jj = j0 + b
        bw = (b + LEAD) % NBUF
        if b < LEAD:
          @pl.when(g > 0)
          def _():
            wait_scatter(bw)
        else:
          wait_scatter(bw)
        @pl.when(jj + LEAD < n_chunks)
        def _():
          pltpu.async_copy(p_hbm.at[sidx.at[jj + LEAD]], gbufs[bw], semg[bw])
        pltpu.make_async_copy(p_hbm.at[sidx.at[0]], gbufs[b], semg[b]).wait()
        pltpu.async_copy(gbufs[b], acc_sh.at[didx.at[jj]], sems[b], add=True)
        if with_count:
          pltpu.async_copy(ones_v, cnt_sh.at[didx.at[jj]], semc[b], add=True)
      return carry

    lax.fori_loop(0, n_chunks // NBUF, group_body, 0)
    for b in range(LEAD, NBUF):
      wait_scatter(b)
    plsc.subcore_barrier()

    pltpu.sync_copy(acc_sh.at[pl.ds(row0, ROWS_PER_TILE)],
                    acc_out.at[pl.ds(row0, ROWS_PER_TILE), pl.ds(cid * H, H)])
    if with_count:
      pltpu.sync_copy(
          cnt_sh.at[pl.ds(row0, ROWS_PER_TILE)],
          cnt_out.at[pl.ds(row0, ROWS_PER_TILE), pl.ds(cid * CNT_W, CNT_W)])

  mesh = plsc.VectorSubcoreMesh(core_axis_name="c", subcore_axis_name="s",
                                num_cores=NC, num_subcores=NS)
  return pl.kernel(body, out_type=out_type, mesh=mesh,
                   scratch_types=scratch,
                   compiler_params=pltpu.CompilerParams(
                       use_tc_tiling_on_sc=False))


def _tc_matmul(x_ref, w_ref, o_ref):
  o_ref[...] = jnp.dot(x_ref[...], w_ref[...],
                       preferred_element_type=jnp.float32)


def _tc_mid(acc_ref, cnt_ref, r1_ref, b1_ref, wl_ref, wr_ref,
            p2_ref, r2_ref, inv_ref):
  cnt = cnt_ref[:N_NODES, :CNT_W] + cnt_ref[:N_NODES, CNT_W:2 * CNT_W]
  inv = 1.0 / jnp.maximum(cnt, 1.0)
  inv_ref[...] = inv
  agg = acc_ref[:N_NODES, :H] + acc_ref[:N_NODES, H:]
  z = jnp.maximum(agg * inv[:, :1] + b1_ref[...] + r1_ref[...], 0.0)
  p2_ref[...] = jnp.dot(z, wl_ref[...], preferred_element_type=jnp.float32)
  r2_ref[...] = jnp.dot(z, wr_ref[...], preferred_element_type=jnp.float32)


def _tc_final(acc_ref, inv_ref, r2_ref, b2_ref, out_ref):
  agg = acc_ref[:N_NODES, :H] + acc_ref[:N_NODES, H:]
  out_ref[...] = agg * inv_ref[:, :1] + b2_ref[...] + r2_ref[...]


@jax.jit
def kernel(x, edge_index, W1l, b1l, W1r, W2l, b2l, W2r):
  n_edges = edge_index.shape[1]
  n_chunks = -(-n_edges // (NW * CHUNK))
  n_chunks = -(-n_chunks // NBUF) * NBUF
  e_pad = NW * n_chunks * CHUNK

  src = edge_index[0].astype(jnp.int32)
  dst = edge_index[1].astype(jnp.int32)
  n_dummy = e_pad - n_edges
  pad_src = jnp.asarray(np.arange(n_dummy, dtype=np.int32) %
                        (N_PAD - N_NODES))
  pad_dst = pad_src + N_NODES
  src = jnp.concatenate([src, pad_src]).reshape(NW, n_chunks, CHUNK)
  dst = jnp.concatenate([dst, pad_dst]).reshape(NW, n_chunks, CHUNK)

  zeros64 = jnp.zeros((N_PAD, H), jnp.float32)
  zeros16 = jnp.zeros((N_PAD, CNT_W), jnp.float32)
  ones = jnp.ones((CHUNK, CNT_W), jnp.float32)
  out64 = jax.ShapeDtypeStruct((N_NODES, H), jnp.float32)

  p1 = pl.pallas_call(_tc_matmul, out_shape=out64)(x, W1l.T)
  r1 = pl.pallas_call(_tc_matmul, out_shape=out64)(x, W1r.T)

  sc1 = _make_sc_scatter(n_chunks, with_count=True)
  acc1, cnt = sc1(p1, src, dst, zeros64, ones, zeros16)

  p2, r2, inv = pl.pallas_call(
      _tc_mid,
      out_shape=[out64, out64,
                 jax.ShapeDtypeStruct((N_NODES, CNT_W), jnp.float32)],
  )(acc1, cnt, r1, b1l.reshape(1, H), W2l.T, W2r.T)

  sc2 = _make_sc_scatter(n_chunks, with_count=False)
  (acc2,) = sc2(p2, src, dst, zeros64)

  out = pl.pallas_call(
      _tc_final,
      out_shape=out64,
  )(acc2, inv, r2, b2l.reshape(1, H))

  return out


# speedup vs baseline: 17.8186x; 1.0823x over previous
import jax
import jax.numpy as jnp
from jax import lax
from jax.experimental import pallas as pl
from jax.experimental.pallas import tpu as pltpu
from jax.experimental.pallas import tpu_sc as plsc

N_NODES = 10000
IN_DIM = 128
H = 64

NC = 2
NS = 16
NW = NC * NS
CHUNK = 128
N_PAD = 10112
ROWS_PER_TILE = N_PAD // NS
CNT_W = 16

NBUF = 6
LEAD = 3

GRID = 5
BLK = N_NODES // GRID


def _make_sc_scatter(tchunks, with_count):
  base = tchunks // NW
  extra = tchunks % NW
  n_seq = base % NBUF
  n_pipe = base - n_seq
  out_type = [jax.ShapeDtypeStruct((N_PAD, NC * H), jnp.float32)]
  if with_count:
    out_type.append(jax.ShapeDtypeStruct((N_PAD, 128), jnp.float32))

  scratch = [
      pltpu.VMEM((base + 1, CHUNK), jnp.int32),
      pltpu.VMEM((base + 1, CHUNK), jnp.int32),
      pltpu.VMEM_SHARED((N_PAD, H), jnp.float32),
  ]
  scratch += [pltpu.VMEM((CHUNK, H), jnp.float32)] * NBUF
  scratch += [pltpu.SemaphoreType.DMA] * NBUF
  scratch += [pltpu.SemaphoreType.DMA] * NBUF
  if with_count:
    scratch += [
        pltpu.VMEM((CHUNK, CNT_W), jnp.float32),
        pltpu.VMEM_SHARED((N_PAD, CNT_W), jnp.float32),
    ]
    scratch += [pltpu.SemaphoreType.DMA] * NBUF

  def body(p_hbm, ei_hbm, zeros64, *rest):
    if with_count:
      ones_hbm, acc_out, cnt_out = rest[:3]
      rest = rest[3:]
    else:
      acc_out = rest[0]
      rest = rest[1:]
    sidx, didx, acc_sh = rest[:3]
    gbufs = rest[3:3 + NBUF]
    semg = rest[3 + NBUF:3 + 2 * NBUF]
    sems = rest[3 + 2 * NBUF:3 + 3 * NBUF]
    if with_count:
      ones_v, cnt_sh = rest[3 + 3 * NBUF:3 + 3 * NBUF + 2]
      semc = rest[3 + 3 * NBUF + 2:]
    cid = lax.axis_index("c")
    sid = lax.axis_index("s")
    w = cid * NS + sid
    row0 = sid * ROWS_PER_TILE
    start_w = w * base + lax.min(w, extra)
    has_extra = w < extra

    pltpu.sync_copy(zeros64.at[pl.ds(row0, ROWS_PER_TILE)],
                    acc_sh.at[pl.ds(row0, ROWS_PER_TILE)])
    if with_count:
      pltpu.sync_copy(zeros64.at[pl.ds(row0, ROWS_PER_TILE), pl.ds(0, CNT_W)],
                      cnt_sh.at[pl.ds(row0, ROWS_PER_TILE)])
      pltpu.sync_copy(ones_hbm, ones_v)
    pltpu.sync_copy(ei_hbm.at[0, pl.ds(start_w, base)],
                    sidx.at[pl.ds(0, base)])
    pltpu.sync_copy(ei_hbm.at[1, pl.ds(start_w, base)],
                    didx.at[pl.ds(0, base)])
    @pl.when(has_extra)
    def _():
      pltpu.sync_copy(ei_hbm.at[0, pl.ds(start_w + base, 1)],
                      sidx.at[pl.ds(base, 1)])
      pltpu.sync_copy(ei_hbm.at[1, pl.ds(start_w + base, 1)],
                      didx.at[pl.ds(base, 1)])
    plsc.subcore_barrier()

    def wait_scatter(slot):
      pltpu.make_async_copy(gbufs[slot], acc_sh.at[didx.at[0]],
                            sems[slot]).wait()
      if with_count:
        pltpu.make_async_copy(ones_v, cnt_sh.at[didx.at[0]],
                              semc[slot]).wait()

    for b in range(LEAD):
      pltpu.async_copy(p_hbm.at[sidx.at[b]], gbufs[b], semg[b])

    def group_body(g, carry):
      j0 = g * NBUF
      for b in range(NBUF):
        jj = j0 + b
        bw = (b + LEAD) % NBUF
        if b < LEAD:
          @pl.when(g > 0)
          def _():
            wait_scatter(bw)
        else:
          wait_scatter(bw)
        @pl.when(jj + LEAD < n_pipe)
        def _():
          pltpu.async_copy(p_hbm.at[sidx.at[jj + LEAD]], gbufs[bw], semg[bw])
        pltpu.make_async_copy(p_hbm.at[sidx.at[0]], gbufs[b], semg[b]).wait()
        pltpu.async_copy(gbufs[b], acc_sh.at[didx.at[jj]], sems[b], add=True)
        if with_count:
          pltpu.async_copy(ones_v, cnt_sh.at[didx.at[jj]], semc[b], add=True)
      return carry

    lax.fori_loop(0, n_pipe // NBUF, group_body, 0)
    for b in range(LEAD, NBUF):
      wait_scatter(b)

    def run_chunk(j):
      pltpu.async_copy(p_hbm.at[sidx.at[j]], gbufs[0], semg[0]).wait()
      pltpu.sync_copy(gbufs[0], acc_sh.at[didx.at[j]], add=True)
      if with_count:
        pltpu.sync_copy(ones_v, cnt_sh.at[didx.at[j]], add=True)

    for j in range(n_pipe, base):
      run_chunk(j)
    @pl.when(has_extra)
    def _():
      run_chunk(base)
    plsc.subcore_barrier()

    pltpu.sync_copy(acc_sh.at[pl.ds(row0, ROWS_PER_TILE)],
                    acc_out.at[pl.ds(row0, ROWS_PER_TILE), pl.ds(cid * H, H)])
    if with_count:
      pltpu.sync_copy(
          cnt_sh.at[pl.ds(row0, ROWS_PER_TILE)],
          cnt_out.at[pl.ds(row0, ROWS_PER_TILE), pl.ds(cid * CNT_W, CNT_W)])

  mesh = plsc.VectorSubcoreMesh(core_axis_name="c", subcore_axis_name="s",
                                num_cores=NC, num_subcores=NS)
  return pl.kernel(body, out_type=out_type, mesh=mesh,
                   scratch_types=scratch,
                   compiler_params=pltpu.CompilerParams(
                       use_tc_tiling_on_sc=False))


def _tc_matmul(x_ref, w_ref, o_ref):
  o_ref[...] = jnp.dot(x_ref[...], w_ref[...],
                       preferred_element_type=jnp.float32)


def _tc_mid(acc_ref, cnt_ref, r1_ref, b1_ref, wl_ref, wr_ref,
            p2_ref, r2_ref, inv_ref):
  cnt = cnt_ref[:, :CNT_W] + cnt_ref[:, CNT_W:2 * CNT_W]
  inv = 1.0 / jnp.maximum(cnt, 1.0)
  inv_ref[...] = inv
  agg = acc_ref[:, :H] + acc_ref[:, H:]
  z = jnp.maximum(agg * inv[:, :1] + b1_ref[...] + r1_ref[...], 0.0)
  p2_ref[...] = jnp.dot(z, wl_ref[...], preferred_element_type=jnp.float32)
  r2_ref[...] = jnp.dot(z, wr_ref[...], preferred_element_type=jnp.float32)


def _tc_final(acc_ref, inv_ref, r2_ref, b2_ref, out_ref):
  agg = acc_ref[:, :H] + acc_ref[:, H:]
  out_ref[...] = agg * inv_ref[:, :1] + b2_ref[...] + r2_ref[...]


def _rows(i):
  return (i, 0)


def _rep(i):
  return (0, 0)


@jax.jit
def kernel(x, edge_index, W1l, b1l, W1r, W2l, b2l, W2r):
  n_edges = edge_index.shape[1]
  tchunks = n_edges // CHUNK
  ei = edge_index.astype(jnp.int32).reshape(2, tchunks, CHUNK)

  zeros64 = jnp.zeros((N_PAD, H), jnp.float32)
  ones = jnp.ones((CHUNK, CNT_W), jnp.float32)
  out64 = jax.ShapeDtypeStruct((N_NODES, H), jnp.float32)

  mm = pl.pallas_call(
      _tc_matmul,
      grid=(GRID,),
      in_specs=[pl.BlockSpec((BLK, IN_DIM), _rows),
                pl.BlockSpec((IN_DIM, H), _rep)],
      out_specs=pl.BlockSpec((BLK, H), _rows),
      out_shape=out64,
  )
  p1 = mm(x, W1l.T)
  r1 = mm(x, W1r.T)

  sc1 = _make_sc_scatter(tchunks, with_count=True)
  acc1, cnt = sc1(p1, ei, zeros64, ones)

  p2, r2, inv = pl.pallas_call(
      _tc_mid,
      grid=(GRID,),
      in_specs=[pl.BlockSpec((BLK, NC * H), _rows),
                pl.BlockSpec((BLK, 128), _rows),
                pl.BlockSpec((BLK, H), _rows),
                pl.BlockSpec((1, H), _rep),
                pl.BlockSpec((H, H), _rep),
                pl.BlockSpec((H, H), _rep)],
      out_specs=[pl.BlockSpec((BLK, H), _rows),
                 pl.BlockSpec((BLK, H), _rows),
                 pl.BlockSpec((BLK, CNT_W), _rows)],
      out_shape=[out64, out64,
                 jax.ShapeDtypeStruct((N_NODES, CNT_W), jnp.float32)],
  )(acc1, cnt, r1, b1l.reshape(1, H), W2l.T, W2r.T)

  sc2 = _make_sc_scatter(tchunks, with_count=False)
  (acc2,) = sc2(p2, ei, zeros64)

  out = pl.pallas_call(
      _tc_final,
      grid=(GRID,),
      in_specs=[pl.BlockSpec((BLK, NC * H), _rows),
                pl.BlockSpec((BLK, CNT_W), _rows),
                pl.BlockSpec((BLK, H), _rows),
                pl.BlockSpec((1, H), _rep)],
      out_specs=pl.BlockSpec((BLK, H), _rows),
      out_shape=out64,
  )(acc2, inv, r2, b2l.reshape(1, H))

  return out
